# baseline (speedup 1.0000x reference)
import functools

import jax
import jax.numpy as jnp
from jax import lax
from jax.experimental import pallas as pl
from jax.experimental.pallas import tpu as pltpu
from jax.experimental.pallas import tpu_sc as plsc

N = 10000
E = 320000
D = 128
H1 = 8
C1 = 16
HID = H1 * C1
OUT = 40
OUTP = 48

NC = 2
NS = 16
NW = NC * NS
NP = 10112
RPT = NP // NS

ACC1W = 144
ACC2W = 64

BN = 632
GRID = NP // BN

GPT = 627
EPT = GPT * 16
EPAD = NW * EPT

_mesh = lambda: plsc.VectorSubcoreMesh(
    core_axis_name="c", subcore_axis_name="s", num_cores=NC, num_subcores=NS)
_params = lambda: pltpu.CompilerParams(
    needs_layout_passes=False, use_tc_tiling_on_sc=False)


def _iota16():
    return lax.broadcasted_iota(jnp.int32, (16,), 0)


def _sc_fused_body(nheads, ch, width, accw, scale,
                   pk_hbm, q_hbm, k_hbm, v_hbm, zeros_hbm,
                   out0_hbm, out1_hbm,
                   acc_sh, idxv,
                   qb0, kb0, vb0, st0,
                   qb1, kb1, vb1, st1,
                   qb2, kb2, vb2, st2,
                   sq0, sk0, sv0, sc0,
                   sq1, sk1, sv1, sc1,
                   sq2, sk2, sv2, sc2):
    cid = lax.axis_index("c")
    sid = lax.axis_index("s")
    wid = sid * NC + cid

    pltpu.sync_copy(zeros_hbm.at[pl.ds(sid * RPT, RPT)],
                    acc_sh.at[pl.ds(sid * RPT, RPT)])
    plsc.subcore_barrier()

    zero16 = jnp.zeros((16,), jnp.float32)
    for stz in (st0, st1, st2):
        for i in range(16):
            stz[i, pl.ds(width, 16)] = zero16

    pltpu.sync_copy(pk_hbm.at[pl.ds(wid * EPT, EPT)], idxv)

    slots = ((qb0, kb0, vb0, st0, sq0, sk0, sv0, sc0),
             (qb1, kb1, vb1, st1, sq1, sk1, sv1, sc1),
             (qb2, kb2, vb2, st2, sq2, sk2, sv2, sc2))

    def unpack(g):
        p = idxv[pl.ds(g * 16, 16)]
        return p & 16383, p >> 14

    def issue(g, s):
        qb, kb, vb, st, sq, sk, sv, sc = s
        srcv, dstv = unpack(g)
        pltpu.async_copy(q_hbm.at[dstv], qb, sq)
        pltpu.async_copy(k_hbm.at[srcv], kb, sk)
        pltpu.async_copy(v_hbm.at[srcv], vb, sv)

    def process(g, s, first):
        qb, kb, vb, st, sq, sk, sv, sc = s
        drain = _iota16()
        pltpu.make_async_copy(q_hbm.at[drain], qb, sq).wait()
        pltpu.make_async_copy(k_hbm.at[drain], kb, sk).wait()
        pltpu.make_async_copy(v_hbm.at[drain], vb, sv).wait()
        if not first:
            pltpu.make_async_copy(st, acc_sh.at[drain], sc).wait()
        rows = _iota16()
        for h in range(nheads):
            acc = jnp.zeros((16,), jnp.float32)
            for c in range(ch):
                col = jnp.full((16,), h * ch + c, jnp.int32)
                acc = acc + (plsc.load_gather(qb, [rows, col])
                             * plsc.load_gather(kb, [rows, col]))
            ex = jnp.exp(jnp.clip(acc * scale, -75.0, 75.0))
            for c in range(ch):
                col = jnp.full((16,), h * ch + c, jnp.int32)
                plsc.store_scatter(
                    st, [rows, col], plsc.load_gather(vb, [rows, col]) * ex)
            plsc.store_scatter(
                st, [rows, jnp.full((16,), width + h, jnp.int32)], ex)
        _, dstv = unpack(g)
        pltpu.async_copy(st, acc_sh.at[dstv], sc, add=True)

    for s in range(3):
        issue(s, slots[s])
    for s in range(3):
        process(s, slots[s], True)
        issue(s + 3, slots[s])

    def body(g3, carry):
        for s in range(3):
            g = 3 * g3 + s
            process(g, slots[s], False)
            issue(g + 3, slots[s])
        return carry

    lax.fori_loop(1, (GPT // 3) - 1, body, 0)

    for s in range(3):
        process(GPT - 3 + s, slots[s], False)
    for s in range(3):
        qb, kb, vb, st, sq, sk, sv, sc = slots[s]
        pltpu.make_async_copy(st, acc_sh.at[_iota16()], sc).wait()

    plsc.subcore_barrier()

    @pl.when(cid == 0)
    def _():
        pltpu.sync_copy(acc_sh.at[pl.ds(sid * RPT, RPT)],
                        out0_hbm.at[pl.ds(sid * RPT, RPT)])

    @pl.when(cid == 1)
    def _():
        pltpu.sync_copy(acc_sh.at[pl.ds(sid * RPT, RPT)],
                        out1_hbm.at[pl.ds(sid * RPT, RPT)])


def _make_sc_fused(nheads, ch, width, accw, scale):
    G = 16
    bufs = []
    for _ in range(3):
        bufs += [pltpu.VMEM((G, width), jnp.float32)] * 3
        bufs += [pltpu.VMEM((G, accw), jnp.float32)]
    return pl.kernel(
        functools.partial(_sc_fused_body, nheads, ch, width, accw, scale),
        out_type=(
            jax.ShapeDtypeStruct((NP, accw), jnp.float32),
            jax.ShapeDtypeStruct((NP, accw), jnp.float32),
        ),
        mesh=_mesh(),
        compiler_params=_params(),
        scratch_types=(
            [pltpu.VMEM_SHARED((NP, accw), jnp.float32),
             pltpu.VMEM((EPT,), jnp.int32)]
            + bufs
            + [pltpu.SemaphoreType.DMA] * 12
        ),
    )


def _proj1_body(x_ref, wq, bq, wk, bk, wv, bv, ws, bs, qo, ko, vo, ro):
    xb = x_ref[...]
    qo[...] = jnp.dot(xb, wq[...], preferred_element_type=jnp.float32) + bq[...]
    ko[...] = jnp.dot(xb, wk[...], preferred_element_type=jnp.float32) + bk[...]
    vo[...] = jnp.dot(xb, wv[...], preferred_element_type=jnp.float32) + bv[...]
    ro[...] = jnp.dot(xb, ws[...], preferred_element_type=jnp.float32) + bs[...]


def _tc_proj1(x, wq, bq, wk, bk, wv, bv, ws, bs):
    full = lambda r, c: pl.BlockSpec((r, c), lambda i: (0, 0))
    blk = lambda c: pl.BlockSpec((BN, c), lambda i: (i, 0))
    return pl.pallas_call(
        _proj1_body,
        grid=(GRID,),
        in_specs=[blk(D), full(D, HID), full(1, HID), full(D, HID), full(1, HID),
                  full(D, HID), full(1, HID), full(D, HID), full(1, HID)],
        out_specs=[blk(HID)] * 4,
        out_shape=[jax.ShapeDtypeStruct((NP, HID), jnp.float32)] * 4,
    )(x, wq, bq, wk, bk, wv, bv, ws, bs)


def _mid_body(a0, a1, r_ref, sel, wbo, wbr, wq, bq, wk, bk, wv, bv, ws, bs,
              q2o, k2o, v2o, r2o):
    a = a0[...] + a1[...]
    num = a[:, :HID]
    den = a[:, HID:HID + H1]
    inv = jnp.where(den > 0.0, 1.0 / den, 0.0)
    invx = jnp.dot(inv, sel[...], preferred_element_type=jnp.float32)
    attn = num * invx
    r = r_ref[...]
    gl = (jnp.dot(attn, wbo[...], preferred_element_type=jnp.float32)
          + jnp.dot(r, wbr[...], preferred_element_type=jnp.float32))
    g = jax.nn.sigmoid(gl[:, 0:1])
    h = g * r + (1.0 - g) * attn
    h = jnp.where(h > 0.0, h, jnp.exp(jnp.minimum(h, 0.0)) - 1.0)
    q2o[...] = jnp.dot(h, wq[...], preferred_element_type=jnp.float32) + bq[...]
    k2o[...] = jnp.dot(h, wk[...], preferred_element_type=jnp.float32) + bk[...]
    v2o[...] = jnp.dot(h, wv[...], preferred_element_type=jnp.float32) + bv[...]
    r2o[...] = jnp.dot(h, ws[...], preferred_element_type=jnp.float32) + bs[...]


def _tc_mid(a0, a1, r1, sel, wbo, wbr, wq, bq, wk, bk, wv, bv, ws, bs):
    full = lambda r, c: pl.BlockSpec((r, c), lambda i: (0, 0))
    blk = lambda c: pl.BlockSpec((BN, c), lambda i: (i, 0))
    return pl.pallas_call(
        _mid_body,
        grid=(GRID,),
        in_specs=[blk(ACC1W), blk(ACC1W), blk(HID), full(H1, HID),
                  full(HID, 8), full(HID, 8),
                  full(HID, OUTP), full(1, OUTP), full(HID, OUTP), full(1, OUTP),
                  full(HID, OUTP), full(1, OUTP), full(HID, OUTP), full(1, OUTP)],
        out_specs=[blk(OUTP)] * 4,
        out_shape=[jax.ShapeDtypeStruct((NP, OUTP), jnp.float32)] * 4,
    )(a0, a1, r1, sel, wbo, wbr, wq, bq, wk, bk, wv, bv, ws, bs)


def _final_body(a0, a1, r_ref, wbo, wbr, fo):
    a = a0[...] + a1[...]
    num = a[:, :OUTP]
    den = a[:, OUTP:OUTP + 1]
    inv = jnp.where(den > 0.0, 1.0 / den, 0.0)
    attn = num * inv
    r = r_ref[...]
    gl = (jnp.dot(attn, wbo[...], preferred_element_type=jnp.float32)
          + jnp.dot(r, wbr[...], preferred_element_type=jnp.float32))
    g = jax.nn.sigmoid(gl[:, 0:1])
    fo[...] = g * r + (1.0 - g) * attn


def _tc_final(a0, a1, r2, wbo, wbr):
    full = lambda r, c: pl.BlockSpec((r, c), lambda i: (0, 0))
    blk = lambda c: pl.BlockSpec((BN, c), lambda i: (i, 0))
    return pl.pallas_call(
        _final_body,
        grid=(GRID,),
        in_specs=[blk(ACC2W), blk(ACC2W), blk(OUTP), full(OUTP, 8), full(OUTP, 8)],
        out_specs=blk(OUTP),
        out_shape=jax.ShapeDtypeStruct((NP, OUTP), jnp.float32),
    )(a0, a1, r2, wbo, wbr)


def kernel(x, edge_index, Wq1, bq1, Wk1, bk1, Wv1, bv1, Ws1, bs1, Wb1,
           Wq2, bq2, Wk2, bk2, Wv2, bv2, Ws2, bs2, Wb2):
    f32 = jnp.float32
    row = lambda b: b.reshape(1, -1)
    pad8 = lambda w: jnp.pad(w, ((0, 0), (0, 7)))
    wbo1 = pad8(Wb1[:HID] + Wb1[2 * HID:])
    wbr1 = pad8(Wb1[HID:2 * HID] - Wb1[2 * HID:])
    wbo2 = jnp.pad(Wb2[:OUT] + Wb2[2 * OUT:], ((0, OUTP - OUT), (0, 7)))
    wbr2 = jnp.pad(Wb2[OUT:2 * OUT] - Wb2[2 * OUT:], ((0, OUTP - OUT), (0, 7)))
    padw = lambda w: jnp.pad(w, ((0, 0), (0, OUTP - OUT)))
    padb = lambda b: jnp.pad(b, (0, OUTP - OUT)).reshape(1, -1)
    sel = (jnp.arange(HID)[None, :] // C1 ==
           jnp.arange(H1)[:, None]).astype(f32)

    pad_e = EPAD - E
    src_p = jnp.concatenate([edge_index[0], jnp.zeros((pad_e,), jnp.int32)])
    dst_p = jnp.concatenate([edge_index[1], jnp.full((pad_e,), N, jnp.int32)])
    packed = src_p + dst_p * 16384
    xp = jnp.pad(x, ((0, NP - N), (0, 0)))
    q1, k1, v1, r1 = _tc_proj1(xp, Wq1, row(bq1), Wk1, row(bk1),
                               Wv1, row(bv1), Ws1, row(bs1))

    z1 = jnp.zeros((NP, ACC1W), f32)
    acc10, acc11 = _make_sc_fused(H1, C1, HID, ACC1W, 1.0 / float(C1) ** 0.5)(
        packed, q1, k1, v1, z1)

    q2, k2, v2, r2 = _tc_mid(acc10, acc11, r1, sel, wbo1, wbr1,
                             padw(Wq2), padb(bq2), padw(Wk2), padb(bk2),
                             padw(Wv2), padb(bv2), padw(Ws2), padb(bs2))

    z2 = jnp.zeros((NP, ACC2W), f32)
    acc20, acc21 = _make_sc_fused(1, OUTP, OUTP, ACC2W, 1.0 / float(OUT) ** 0.5)(
        packed, q2, k2, v2, z2)

    out = _tc_final(acc20, acc21, r2, wbo2, wbr2)
    return out[:N, :OUT]


# speedup vs baseline: 1.6083x; 1.6083x over previous
import functools

import jax
import jax.numpy as jnp
from jax import lax
from jax.experimental import pallas as pl
from jax.experimental.pallas import tpu as pltpu
from jax.experimental.pallas import tpu_sc as plsc

N = 10000
E = 320000
D = 128
H1 = 8
C1 = 16
HID = H1 * C1
OUT = 40
OUTP = 48

NC = 2
NS = 16
NW = NC * NS
NP = 10112
RPT = NP // NS

ACC1W = 144
ACC2W = 64

BN = 632
GRID = NP // BN

GPT = 627
EPT = GPT * 16
EPAD = NW * EPT

_mesh = lambda: plsc.VectorSubcoreMesh(
    core_axis_name="c", subcore_axis_name="s", num_cores=NC, num_subcores=NS)
_params = lambda: pltpu.CompilerParams(
    needs_layout_passes=False, use_tc_tiling_on_sc=False)


def _iota16():
    return lax.broadcasted_iota(jnp.int32, (16,), 0)


def _sc_fused_body(nheads, ch, width, accw, scale,
                   pk_hbm, q_hbm, k_hbm, v_hbm, zeros_hbm,
                   out0_hbm, out1_hbm,
                   acc_sh, idxv,
                   qb0, kb0, vb0, st0,
                   qb1, kb1, vb1, st1,
                   qb2, kb2, vb2, st2,
                   exbuf,
                   sq0, sk0, sv0, sc0,
                   sq1, sk1, sv1, sc1,
                   sq2, sk2, sv2, sc2):
    cid = lax.axis_index("c")
    sid = lax.axis_index("s")
    wid = sid * NC + cid

    pltpu.sync_copy(zeros_hbm.at[pl.ds(sid * RPT, RPT)],
                    acc_sh.at[pl.ds(sid * RPT, RPT)])
    plsc.subcore_barrier()

    zero16 = jnp.zeros((16,), jnp.float32)
    for i in range(16):
        exbuf[i, :] = zero16

    pltpu.sync_copy(pk_hbm.at[pl.ds(wid * EPT, EPT)], idxv)

    slots = ((qb0, kb0, vb0, st0, sq0, sk0, sv0, sc0),
             (qb1, kb1, vb1, st1, sq1, sk1, sv1, sc1),
             (qb2, kb2, vb2, st2, sq2, sk2, sv2, sc2))

    def unpack(g):
        p = idxv[pl.ds(g * 16, 16)]
        return p & 16383, p >> 14

    def issue(g, s):
        qb, kb, vb, st, sq, sk, sv, sc = s
        srcv, dstv = unpack(g)
        pltpu.async_copy(q_hbm.at[dstv], qb, sq)
        pltpu.async_copy(k_hbm.at[srcv], kb, sk)
        pltpu.async_copy(v_hbm.at[srcv], vb, sv)

    def process(g, s, first):
        qb, kb, vb, st, sq, sk, sv, sc = s
        drain = _iota16()
        pltpu.make_async_copy(q_hbm.at[drain], qb, sq).wait()
        pltpu.make_async_copy(k_hbm.at[drain], kb, sk).wait()
        rows = _iota16()
        for h in range(nheads):
            acc = jnp.zeros((16,), jnp.float32)
            for c in range(ch):
                col = jnp.full((16,), h * ch + c, jnp.int32)
                acc = acc + (plsc.load_gather(qb, [rows, col])
                             * plsc.load_gather(kb, [rows, col]))
            ex = jnp.exp(jnp.clip(acc * scale, -75.0, 75.0))
            if nheads == 1:
                exbuf[0, :] = ex
            else:
                plsc.store_scatter(
                    exbuf, [rows, jnp.full((16,), h, jnp.int32)], ex)
        pltpu.make_async_copy(v_hbm.at[drain], vb, sv).wait()
        if not first:
            pltpu.make_async_copy(st, acc_sh.at[drain], sc).wait()
        exv0 = exbuf[0, :]
        for i in range(16):
            if nheads == 1:
                s_ = exv0[i]
                for j in range(width // 16):
                    st[i, pl.ds(j * 16, 16)] = vb[i, pl.ds(j * 16, 16)] * s_
                st[i, pl.ds(width, 16)] = jnp.where(_iota16() == 0, s_, 0.0)
            else:
                erow = exbuf[i, :]
                st[i, pl.ds(width, 16)] = erow
                for j in range(nheads):
                    s_ = erow[j]
                    st[i, pl.ds(j * ch, ch)] = vb[i, pl.ds(j * ch, ch)] * s_
        _, dstv = unpack(g)
        pltpu.async_copy(st, acc_sh.at[dstv], sc, add=True)

    for s in range(3):
        issue(s, slots[s])
    for s in range(3):
        process(s, slots[s], True)
        issue(s + 3, slots[s])

    def body(g3, carry):
        for s in range(3):
            g = 3 * g3 + s
            process(g, slots[s], False)
            issue(g + 3, slots[s])
        return carry

    lax.fori_loop(1, (GPT // 3) - 1, body, 0)

    for s in range(3):
        process(GPT - 3 + s, slots[s], False)
    for s in range(3):
        qb, kb, vb, st, sq, sk, sv, sc = slots[s]
        pltpu.make_async_copy(st, acc_sh.at[_iota16()], sc).wait()

    plsc.subcore_barrier()

    @pl.when(cid == 0)
    def _():
        pltpu.sync_copy(acc_sh.at[pl.ds(sid * RPT, RPT)],
                        out0_hbm.at[pl.ds(sid * RPT, RPT)])

    @pl.when(cid == 1)
    def _():
        pltpu.sync_copy(acc_sh.at[pl.ds(sid * RPT, RPT)],
                        out1_hbm.at[pl.ds(sid * RPT, RPT)])


def _make_sc_fused(nheads, ch, width, accw, scale):
    G = 16
    bufs = []
    for _ in range(3):
        bufs += [pltpu.VMEM((G, width), jnp.float32)] * 3
        bufs += [pltpu.VMEM((G, accw), jnp.float32)]
    return pl.kernel(
        functools.partial(_sc_fused_body, nheads, ch, width, accw, scale),
        out_type=(
            jax.ShapeDtypeStruct((NP, accw), jnp.float32),
            jax.ShapeDtypeStruct((NP, accw), jnp.float32),
        ),
        mesh=_mesh(),
        compiler_params=_params(),
        scratch_types=(
            [pltpu.VMEM_SHARED((NP, accw), jnp.float32),
             pltpu.VMEM((EPT,), jnp.int32)]
            + bufs
            + [pltpu.VMEM((16, 16), jnp.float32)]
            + [pltpu.SemaphoreType.DMA] * 12
        ),
    )


def _proj1_body(x_ref, wq, bq, wk, bk, wv, bv, ws, bs, qo, ko, vo, ro):
    xb = x_ref[...]
    qo[...] = jnp.dot(xb, wq[...], preferred_element_type=jnp.float32) + bq[...]
    ko[...] = jnp.dot(xb, wk[...], preferred_element_type=jnp.float32) + bk[...]
    vo[...] = jnp.dot(xb, wv[...], preferred_element_type=jnp.float32) + bv[...]
    ro[...] = jnp.dot(xb, ws[...], preferred_element_type=jnp.float32) + bs[...]


def _tc_proj1(x, wq, bq, wk, bk, wv, bv, ws, bs):
    full = lambda r, c: pl.BlockSpec((r, c), lambda i: (0, 0))
    blk = lambda c: pl.BlockSpec((BN, c), lambda i: (i, 0))
    return pl.pallas_call(
        _proj1_body,
        grid=(GRID,),
        in_specs=[blk(D), full(D, HID), full(1, HID), full(D, HID), full(1, HID),
                  full(D, HID), full(1, HID), full(D, HID), full(1, HID)],
        out_specs=[blk(HID)] * 4,
        out_shape=[jax.ShapeDtypeStruct((NP, HID), jnp.float32)] * 4,
    )(x, wq, bq, wk, bk, wv, bv, ws, bs)


def _mid_body(a0, a1, r_ref, sel, wbo, wbr, wq, bq, wk, bk, wv, bv, ws, bs,
              q2o, k2o, v2o, r2o):
    a = a0[...] + a1[...]
    num = a[:, :HID]
    den = a[:, HID:HID + H1]
    inv = jnp.where(den > 0.0, 1.0 / den, 0.0)
    invx = jnp.dot(inv, sel[...], preferred_element_type=jnp.float32)
    attn = num * invx
    r = r_ref[...]
    gl = (jnp.dot(attn, wbo[...], preferred_element_type=jnp.float32)
          + jnp.dot(r, wbr[...], preferred_element_type=jnp.float32))
    g = jax.nn.sigmoid(gl[:, 0:1])
    h = g * r + (1.0 - g) * attn
    h = jnp.where(h > 0.0, h, jnp.exp(jnp.minimum(h, 0.0)) - 1.0)
    q2o[...] = jnp.dot(h, wq[...], preferred_element_type=jnp.float32) + bq[...]
    k2o[...] = jnp.dot(h, wk[...], preferred_element_type=jnp.float32) + bk[...]
    v2o[...] = jnp.dot(h, wv[...], preferred_element_type=jnp.float32) + bv[...]
    r2o[...] = jnp.dot(h, ws[...], preferred_element_type=jnp.float32) + bs[...]


def _tc_mid(a0, a1, r1, sel, wbo, wbr, wq, bq, wk, bk, wv, bv, ws, bs):
    full = lambda r, c: pl.BlockSpec((r, c), lambda i: (0, 0))
    blk = lambda c: pl.BlockSpec((BN, c), lambda i: (i, 0))
    return pl.pallas_call(
        _mid_body,
        grid=(GRID,),
        in_specs=[blk(ACC1W), blk(ACC1W), blk(HID), full(H1, HID),
                  full(HID, 8), full(HID, 8),
                  full(HID, OUTP), full(1, OUTP), full(HID, OUTP), full(1, OUTP),
                  full(HID, OUTP), full(1, OUTP), full(HID, OUTP), full(1, OUTP)],
        out_specs=[blk(OUTP)] * 4,
        out_shape=[jax.ShapeDtypeStruct((NP, OUTP), jnp.float32)] * 4,
    )(a0, a1, r1, sel, wbo, wbr, wq, bq, wk, bk, wv, bv, ws, bs)


def _final_body(a0, a1, r_ref, wbo, wbr, fo):
    a = a0[...] + a1[...]
    num = a[:, :OUTP]
    den = a[:, OUTP:OUTP + 1]
    inv = jnp.where(den > 0.0, 1.0 / den, 0.0)
    attn = num * inv
    r = r_ref[...]
    gl = (jnp.dot(attn, wbo[...], preferred_element_type=jnp.float32)
          + jnp.dot(r, wbr[...], preferred_element_type=jnp.float32))
    g = jax.nn.sigmoid(gl[:, 0:1])
    fo[...] = g * r + (1.0 - g) * attn


def _tc_final(a0, a1, r2, wbo, wbr):
    full = lambda r, c: pl.BlockSpec((r, c), lambda i: (0, 0))
    blk = lambda c: pl.BlockSpec((BN, c), lambda i: (i, 0))
    return pl.pallas_call(
        _final_body,
        grid=(GRID,),
        in_specs=[blk(ACC2W), blk(ACC2W), blk(OUTP), full(OUTP, 8), full(OUTP, 8)],
        out_specs=blk(OUTP),
        out_shape=jax.ShapeDtypeStruct((NP, OUTP), jnp.float32),
    )(a0, a1, r2, wbo, wbr)


def kernel(x, edge_index, Wq1, bq1, Wk1, bk1, Wv1, bv1, Ws1, bs1, Wb1,
           Wq2, bq2, Wk2, bk2, Wv2, bv2, Ws2, bs2, Wb2):
    f32 = jnp.float32
    row = lambda b: b.reshape(1, -1)
    pad8 = lambda w: jnp.pad(w, ((0, 0), (0, 7)))
    wbo1 = pad8(Wb1[:HID] + Wb1[2 * HID:])
    wbr1 = pad8(Wb1[HID:2 * HID] - Wb1[2 * HID:])
    wbo2 = jnp.pad(Wb2[:OUT] + Wb2[2 * OUT:], ((0, OUTP - OUT), (0, 7)))
    wbr2 = jnp.pad(Wb2[OUT:2 * OUT] - Wb2[2 * OUT:], ((0, OUTP - OUT), (0, 7)))
    padw = lambda w: jnp.pad(w, ((0, 0), (0, OUTP - OUT)))
    padb = lambda b: jnp.pad(b, (0, OUTP - OUT)).reshape(1, -1)
    sel = (jnp.arange(HID)[None, :] // C1 ==
           jnp.arange(H1)[:, None]).astype(f32)

    pad_e = EPAD - E
    src_p = jnp.concatenate([edge_index[0], jnp.zeros((pad_e,), jnp.int32)])
    dst_p = jnp.concatenate([edge_index[1], jnp.full((pad_e,), N, jnp.int32)])
    packed = src_p + dst_p * 16384
    xp = jnp.pad(x, ((0, NP - N), (0, 0)))
    q1, k1, v1, r1 = _tc_proj1(xp, Wq1, row(bq1), Wk1, row(bk1),
                               Wv1, row(bv1), Ws1, row(bs1))

    z1 = jnp.zeros((NP, ACC1W), f32)
    acc10, acc11 = _make_sc_fused(H1, C1, HID, ACC1W, 1.0 / float(C1) ** 0.5)(
        packed, q1, k1, v1, z1)

    q2, k2, v2, r2 = _tc_mid(acc10, acc11, r1, sel, wbo1, wbr1,
                             padw(Wq2), padb(bq2), padw(Wk2), padb(bk2),
                             padw(Wv2), padb(bv2), padw(Ws2), padb(bs2))

    z2 = jnp.zeros((NP, ACC2W), f32)
    acc20, acc21 = _make_sc_fused(1, OUTP, OUTP, ACC2W, 1.0 / float(OUT) ** 0.5)(
        packed, q2, k2, v2, z2)

    out = _tc_final(acc20, acc21, r2, wbo2, wbr2)
    return out[:N, :OUT]
